# baseline (speedup 1.0000x reference)
import functools

import jax
import jax.numpy as jnp
from jax import lax
from jax.experimental import pallas as pl
from jax.experimental.pallas import tpu as pltpu
from jax.experimental.pallas import tpu_sc as plsc

B = 16384
D = 64
SLAB = 8
NSLAB = 1000000 // SLAB
NUM_CORES = 2
NUM_SUBCORES = 16
NW = NUM_CORES * NUM_SUBCORES
BPW = B // NW
IC = 4
ICHUNK = BPW // IC
LANES = 16
NR = BPW // LANES


def _body(tu_hbm, ti_hbm, u_hbm, i_hbm, out_hbm,
          tidx_v, sub_v, ubuf_v, ibuf_v, out_v, sem_u, sem_i):
    wid = lax.axis_index("s") * NUM_CORES + lax.axis_index("c")
    base = wid * BPW

    for t, idx_hbm in ((0, tu_hbm), (1, ti_hbm)):
        for j in range(IC):
            pltpu.sync_copy(idx_hbm.at[pl.ds(base + j * ICHUNK, ICHUNK)],
                            tidx_v.at[t, pl.ds(j * ICHUNK, ICHUNK)])
    for t in range(2):
        for k in range(BPW // LANES):
            flat = k * LANES
            v = tidx_v[t, pl.ds(flat, LANES)]
            sub_v[t, pl.ds(flat, LANES)] = lax.bitwise_and(v, 7)
            tidx_v[t, pl.ds(flat, LANES)] = lax.shift_right_logical(v, 3)
    lane = lax.iota(jnp.int32, LANES)

    def fire_round(r, parity):
        uslabs = tidx_v[0, pl.ds(r * LANES, LANES)]
        islabs = tidx_v[1, pl.ds(r * LANES, LANES)]
        for k in range(LANES):
            pltpu.async_copy(u_hbm.at[uslabs[k]], ubuf_v.at[parity, k], sem_u)
            pltpu.async_copy(i_hbm.at[islabs[k]], ibuf_v.at[parity, k], sem_i)

    def drain_round(parity):
        for k in range(LANES):
            pltpu.make_async_copy(u_hbm.at[0], ubuf_v.at[parity, k],
                                  sem_u).wait()
            pltpu.make_async_copy(i_hbm.at[0], ibuf_v.at[parity, k],
                                  sem_i).wait()

    def compute_round(r, parity):
        pvec = jnp.full((LANES,), parity, jnp.int32)
        usub = sub_v[0, pl.ds(r * LANES, LANES)]
        isub = sub_v[1, pl.ds(r * LANES, LANES)]
        acc = jnp.zeros((LANES,), jnp.float32)
        for d in range(D):
            col = jnp.full((LANES,), d, jnp.int32)
            uv = plsc.load_gather(ubuf_v, [pvec, lane, usub, col])
            iv = plsc.load_gather(ibuf_v, [pvec, lane, isub, col])
            acc = acc + uv * iv
        out_v[pl.ds(r * LANES, LANES)] = acc

    fire_round(0, 0)

    def loop_body(r, carry):
        parity = lax.rem(r, 2)
        fire_round(r, parity)
        drain_round(1 - parity)
        compute_round(r - 1, 1 - parity)
        return carry

    lax.fori_loop(1, NR, loop_body, 0)

    drain_round((NR - 1) % 2)
    compute_round(NR - 1, (NR - 1) % 2)

    pltpu.sync_copy(out_v, out_hbm.at[pl.ds(base, BPW)])


@functools.partial(
    pl.kernel,
    out_type=jax.ShapeDtypeStruct((B,), jnp.float32),
    mesh=plsc.VectorSubcoreMesh(core_axis_name="c", subcore_axis_name="s"),
    compiler_params=pltpu.CompilerParams(needs_layout_passes=False),
    scratch_types=[
        pltpu.VMEM((2, BPW), jnp.int32),
        pltpu.VMEM((2, BPW), jnp.int32),
        pltpu.VMEM((2, LANES, SLAB, D), jnp.float32),
        pltpu.VMEM((2, LANES, SLAB, D), jnp.float32),
        pltpu.VMEM((BPW,), jnp.float32),
        pltpu.SemaphoreType.DMA,
        pltpu.SemaphoreType.DMA,
    ],
)
def _mf_sc(tu_hbm, ti_hbm, u_hbm, i_hbm, out_hbm,
           tidx_v, sub_v, ubuf_v, ibuf_v, out_v, sem_u, sem_i):
    _body(tu_hbm, ti_hbm, u_hbm, i_hbm, out_hbm,
          tidx_v, sub_v, ubuf_v, ibuf_v, out_v, sem_u, sem_i)


def kernel(Tu, Ti, uY, iY):
    u3 = uY.reshape(NSLAB, SLAB, D)
    i3 = iY.reshape(NSLAB, SLAB, D)
    return _mf_sc(Tu.astype(jnp.int32), Ti.astype(jnp.int32), u3, i3)


# speedup vs baseline: 1.0049x; 1.0049x over previous
import functools

import jax
import jax.numpy as jnp
from jax import lax
from jax.experimental import pallas as pl
from jax.experimental.pallas import tpu as pltpu
from jax.experimental.pallas import tpu_sc as plsc

B = 16384
D = 64
NT = 1000000
BLK = 128
NFULL = NT // BLK
TAIL0 = NFULL * BLK
NTAIL = NT - TAIL0
NUM_CORES = 2
NUM_SUBCORES = 16
NW = NUM_CORES * NUM_SUBCORES
WBLK = 245
LANES = 16
CHUNK = 2
CW = CHUNK * BLK
NBIN = 256
FROWS = 128
GPF = FROWS // LANES
EMB_ROWS = B + FROWS
BPW = B // NW


def _emit_groups(s, e, gctr, colbase, colmask, buf, bufidx0, emb_hbm,
                 sorted_v, stage_v, sidx_v, sem_emb, lane):
    def group(g, c):
        base = s + g * LANES
        pv = sorted_v[pl.ds(base, LANES)]
        mask = lane < (e - base)
        col = lax.bitwise_and(lax.shift_right_logical(pv, 14) - colbase,
                              colmask)
        fslot = lax.bitwise_and(lax.shift_right_logical(c, 3), 1)
        frow = lax.bitwise_and(c, 7) * LANES
        fslotv = jnp.full((LANES,), fslot, jnp.int32)

        @pl.when(lax.bitwise_and(c, 7) == 0)
        def _():
            @pl.when(c >= 2 * GPF)
            def _():
                pltpu.make_async_copy(emb_hbm.at[pl.ds(0, FROWS)],
                                      stage_v.at[0], sem_emb).wait()
            for q in range(GPF):
                sidx_v[fslot, pl.ds(q * LANES, LANES)] = (
                    B + q * LANES + lane)

        bv = jnp.where(mask, lax.bitwise_and(pv, 0x3FFF), B + frow + lane)
        for d in range(D):
            dv = jnp.full((LANES,), d, jnp.int32)
            vals = plsc.load_gather(buf, [bufidx0, dv, col])
            plsc.store_scatter(stage_v, [fslotv, frow + lane, dv], vals)
        sidx_v[fslot, pl.ds(frow, LANES)] = bv

        @pl.when(lax.bitwise_and(c, 7) == 7)
        def _():
            pltpu.async_copy(stage_v.at[fslot],
                             emb_hbm.at[sidx_v.at[fslot]], sem_emb)

        return c + 1

    ng = lax.div(e - s + LANES - 1, jnp.int32(LANES))
    return lax.fori_loop(0, ng, group, gctr)


def _pass(idx_hbm, tab_hbm, tail_hbm, emb_hbm, wid,
          sidx_stage, hits_v, sorted_v, cnt_v, off_v, woff_v,
          chunk_v, tail_v, stage_v, sidx_v, sem_c, sem_emb):
    lane = lax.iota(jnp.int32, LANES)
    m0 = lane < 1
    lo = wid * WBLK
    nloc = jnp.minimum(lo + WBLK, NFULL + 1) - lo
    nfull = jnp.minimum(lo + WBLK, NFULL) - lo

    with jax.named_scope("stage_idx"):
        for j in range(B // 2048):
            pltpu.sync_copy(idx_hbm.at[pl.ds(j * 2048, 2048)],
                            sidx_stage.at[pl.ds(j * 2048, 2048)])

    def scan_step(st, n):
        v = sidx_stage[pl.ds(st * LANES, LANES)]
        local = lax.shift_right_logical(v, 7) - lo
        m = (local >= 0) & (local < nloc)
        phase = lax.bitwise_and(v, BLK - 1)
        packed = lax.bitwise_or(
            lax.bitwise_or(lax.shift_left(local, 21),
                           lax.shift_left(phase, 14)),
            st * LANES + lane)
        plsc.store_compressed(hits_v.at[pl.ds(n, LANES)], packed, mask=m)
        return n + jnp.max(plsc.all_reduce_population_count(m))

    with jax.named_scope("scan"):
        n = lax.fori_loop(0, B // LANES, scan_step, jnp.int32(0))

    zeros = jnp.zeros((LANES,), jnp.int32)
    for g in range(NBIN // LANES):
        cnt_v[pl.ds(g * LANES, LANES)] = zeros

    def hist_step(h, c):
        pv = plsc.load_gather(hits_v, [jnp.full((LANES,), h, jnp.int32)])
        loc = lax.shift_right_logical(pv, 21)
        cur = plsc.load_gather(cnt_v, [loc])
        plsc.store_scatter(cnt_v, [loc], cur + 1, mask=m0)
        return c

    with jax.named_scope("hist"):
        lax.fori_loop(0, n, hist_step, 0)

    fifteen = jnp.full((LANES,), 15, jnp.int32)
    run = zeros
    for g in range(NBIN // LANES):
        v = cnt_v[pl.ds(g * LANES, LANES)]
        s = plsc.cumsum(v)
        excl = s - v + run
        off_v[pl.ds(g * LANES, LANES)] = excl
        woff_v[pl.ds(g * LANES, LANES)] = excl
        run = run + jnp.take(s, fifteen)

    def place_step(h, c):
        pv = plsc.load_gather(hits_v, [jnp.full((LANES,), h, jnp.int32)])
        loc = lax.shift_right_logical(pv, 21)
        pos = plsc.load_gather(woff_v, [loc])
        plsc.store_scatter(sorted_v, [pos], pv, mask=m0)
        plsc.store_scatter(woff_v, [loc], pos + 1, mask=m0)
        return c

    with jax.named_scope("place"):
        lax.fori_loop(0, n, place_step, 0)

    nc = lax.div(nfull + CHUNK - 1, jnp.int32(CHUNK))

    def chunk_start(c):
        return jnp.minimum(c * CHUNK, nfull - CHUNK)

    def fire(c):
        col0 = pl.multiple_of((lo + chunk_start(c)) * BLK, BLK)
        pltpu.async_copy(tab_hbm.at[:, pl.ds(col0, CW)],
                         chunk_v.at[lax.rem(c, 2)], sem_c)

    def drain_chunk():
        pltpu.make_async_copy(tab_hbm.at[:, pl.ds(0, CW)],
                              chunk_v.at[0], sem_c).wait()

    fire(jnp.int32(0))

    def sweep(c, gctr):
        fire(jnp.minimum(c + 1, nc - 1))
        drain_chunk()
        ofs = chunk_start(c)
        bend = jnp.minimum(ofs + CHUNK, nfull)
        s = jnp.max(plsc.load_gather(
            off_v, [jnp.full((LANES,), ofs, jnp.int32)]))
        e = jnp.max(plsc.load_gather(
            off_v, [jnp.full((LANES,), bend, jnp.int32)]))
        parv = jnp.full((LANES,), lax.rem(c, 2), jnp.int32)
        return _emit_groups(s, e, gctr, ofs * BLK, CW - 1, chunk_v, parv,
                            emb_hbm, sorted_v, stage_v, sidx_v, sem_emb,
                            lane)

    with jax.named_scope("sweep"):
        gctr = lax.fori_loop(0, nc, sweep, jnp.int32(0))
        drain_chunk()

    pltpu.sync_copy(tail_hbm, tail_v.at[0])
    ts = jnp.max(plsc.load_gather(
        off_v, [jnp.full((LANES,), nfull, jnp.int32)]))
    te = jnp.max(plsc.load_gather(
        off_v, [jnp.full((LANES,), nfull + 1, jnp.int32)]))
    zv = jnp.zeros((LANES,), jnp.int32)
    gctr = _emit_groups(ts, te, gctr, nfull * BLK, NTAIL - 1, tail_v, zv,
                        emb_hbm, sorted_v, stage_v, sidx_v, sem_emb, lane)

    fill = lax.bitwise_and(gctr, GPF - 1)

    @pl.when(fill > 0)
    def _():
        fslot = lax.bitwise_and(lax.shift_right_logical(gctr, 3), 1)
        pltpu.async_copy(stage_v.at[fslot],
                         emb_hbm.at[sidx_v.at[fslot]], sem_emb)

    nf = (lax.shift_right_logical(gctr, 3)
          + jnp.where(fill > 0, jnp.int32(1), jnp.int32(0)))

    def drain_emb(k, c):
        pltpu.make_async_copy(emb_hbm.at[pl.ds(0, FROWS)],
                              stage_v.at[0], sem_emb).wait()
        return c

    lax.fori_loop(0, jnp.minimum(nf, 2), drain_emb, 0)


@functools.partial(
    pl.kernel,
    out_type=[jax.ShapeDtypeStruct((EMB_ROWS, 2 * D), jnp.float32),
              jax.ShapeDtypeStruct((EMB_ROWS, 2 * D), jnp.float32)],
    mesh=plsc.VectorSubcoreMesh(core_axis_name="c", subcore_axis_name="s"),
    compiler_params=pltpu.CompilerParams(needs_layout_passes=False),
    scratch_types=[
        pltpu.VMEM((B,), jnp.int32),
        pltpu.VMEM((B + LANES,), jnp.int32),
        pltpu.VMEM((B + LANES,), jnp.int32),
        pltpu.VMEM((NBIN,), jnp.int32),
        pltpu.VMEM((NBIN,), jnp.int32),
        pltpu.VMEM((NBIN,), jnp.int32),
        pltpu.VMEM((2, D, CW), jnp.float32),
        pltpu.VMEM((1, D, NTAIL), jnp.float32),
        pltpu.VMEM((2, FROWS, 2 * D), jnp.float32),
        pltpu.VMEM((2, FROWS), jnp.int32),
        pltpu.SemaphoreType.DMA,
        pltpu.SemaphoreType.DMA,
    ],
)
def _mf_gather(tu_hbm, ti_hbm, u_hbm, i_hbm, utail_hbm, itail_hbm,
               uemb_hbm, iemb_hbm,
               sidx_stage, hits_v, sorted_v, cnt_v, off_v, woff_v,
               chunk_v, tail_v, stage_v, sidx_v, sem_c, sem_emb):
    wid = lax.axis_index("s") * NUM_CORES + lax.axis_index("c")
    _pass(tu_hbm, u_hbm, utail_hbm, uemb_hbm, wid,
          sidx_stage, hits_v, sorted_v, cnt_v, off_v, woff_v,
          chunk_v, tail_v, stage_v, sidx_v, sem_c, sem_emb)
    _pass(ti_hbm, i_hbm, itail_hbm, iemb_hbm, wid,
          sidx_stage, hits_v, sorted_v, cnt_v, off_v, woff_v,
          chunk_v, tail_v, stage_v, sidx_v, sem_c, sem_emb)


@functools.partial(
    pl.kernel,
    out_type=jax.ShapeDtypeStruct((B,), jnp.float32),
    mesh=plsc.VectorSubcoreMesh(core_axis_name="c", subcore_axis_name="s"),
    compiler_params=pltpu.CompilerParams(needs_layout_passes=False),
    scratch_types=[
        pltpu.VMEM((BPW // 2, 2 * D), jnp.float32),
        pltpu.VMEM((BPW // 2, 2 * D), jnp.float32),
        pltpu.VMEM((BPW,), jnp.float32),
    ],
)
def _mf_dot(uemb_hbm, iemb_hbm, out_hbm, ubuf_v, ibuf_v, out_v):
    wid = lax.axis_index("s") * NUM_CORES + lax.axis_index("c")
    base = wid * BPW
    lane = lax.iota(jnp.int32, LANES)
    half = BPW // 2
    for r in range(2):
        pltpu.sync_copy(uemb_hbm.at[pl.ds(base + r * half, half)], ubuf_v)
        pltpu.sync_copy(iemb_hbm.at[pl.ds(base + r * half, half)], ibuf_v)
        for g in range(half // LANES):
            rows = lane + g * LANES
            acc = jnp.zeros((LANES,), jnp.float32)
            for d in range(D):
                dv = jnp.full((LANES,), d, jnp.int32)
                uv = plsc.load_gather(ubuf_v, [rows, dv])
                iv = plsc.load_gather(ibuf_v, [rows, dv])
                acc = acc + uv * iv
            out_v[pl.ds(r * half + g * LANES, LANES)] = acc
    pltpu.sync_copy(out_v, out_hbm.at[pl.ds(base, BPW)])


def kernel(Tu, Ti, uY, iY):
    tu = Tu.astype(jnp.int32)
    ti = Ti.astype(jnp.int32)
    u_t = uY.T
    i_t = iY.T
    u_tail = uY[TAIL0:].T
    i_tail = iY[TAIL0:].T
    u_emb, i_emb = _mf_gather(tu, ti, u_t, i_t, u_tail, i_tail)
    return _mf_dot(u_emb, i_emb)
